# Initial kernel scaffold; baseline (speedup 1.0000x reference)
import jax
import jax.numpy as jnp
from jax.experimental import pallas as pl


def kernel(c1_w, c1_b, c2_w, c2_b, c3_w, c3_b, fc1_w, fc1_b, fc2_w, fc2_b, x_nchw):
    raise NotImplementedError("write your pallas kernel here")


# speedup vs baseline: 37.1275x; 37.1275x over previous
import functools

import jax
import jax.numpy as jnp
from jax.experimental import pallas as pl
from jax.experimental.pallas import tpu as pltpu

_B_BLK = 16


def _conv_stack_kernel(x_ref, w1_ref, b1_ref, w2_ref, b2_ref, w3_ref, b3_ref,
                       o_ref):
    B = x_ref.shape[0]
    x = x_ref[...]

    p = jnp.concatenate(
        [x[:, 0:20, 0:20, :], x[:, 0:20, 1:21, :],
         x[:, 1:21, 0:20, :], x[:, 1:21, 1:21, :]], axis=-1)
    a = jnp.dot(p.reshape(B * 400, 256), w1_ref[...],
                preferred_element_type=jnp.float32)
    a = jnp.maximum(a + b1_ref[...], 0.0).reshape(B, 20, 20, 32)

    a = a.reshape(B, 10, 2, 10, 2, 32).transpose(0, 1, 3, 2, 4, 5)
    a = a.reshape(B, 10, 10, 128)
    p = jnp.concatenate(
        [a[:, 0:9, 0:9, :], a[:, 0:9, 1:10, :],
         a[:, 1:10, 0:9, :], a[:, 1:10, 1:10, :]], axis=-1)
    a = jnp.dot(p.reshape(B * 81, 512), w2_ref[...],
                preferred_element_type=jnp.float32)
    a = jnp.maximum(a + b2_ref[...], 0.0).reshape(B, 9, 9, 64)

    p = jnp.concatenate(
        [a[:, i:i + 7, j:j + 7, :] for i in range(3) for j in range(3)],
        axis=-1)
    a = jnp.dot(p.reshape(B * 49, 576), w3_ref[...],
                preferred_element_type=jnp.float32)
    a = jnp.maximum(a + b3_ref[...], 0.0)
    o_ref[...] = a.reshape(B, 49, 64)


def _fc_head_kernel(x_ref, w1_ref, b1_ref, w2_ref, b2_ref, o_ref):
    h = jnp.dot(x_ref[...], w1_ref[...], preferred_element_type=jnp.float32)
    h = jnp.maximum(h + b1_ref[...], 0.0)
    o = jnp.dot(h, w2_ref[...], preferred_element_type=jnp.float32)
    o_ref[...] = o + b2_ref[...]


def kernel(c1_w, c1_b, c2_w, c2_b, c3_w, c3_b, fc1_w, fc1_b, fc2_w, fc2_b,
           x_nchw):
    N = x_nchw.shape[0]
    x = x_nchw.astype(jnp.float32)

    xs = x.reshape(N, 4, 21, 4, 21, 4).transpose(0, 2, 4, 1, 3, 5)
    xs = xs.reshape(N, 21, 21, 64)

    w1p = c1_w.reshape(2, 4, 2, 4, 4, 32).transpose(0, 2, 4, 1, 3, 5)
    w1p = w1p.reshape(256, 32)
    w2p = c2_w.reshape(2, 2, 2, 2, 32, 64).transpose(0, 2, 1, 3, 4, 5)
    w2p = w2p.reshape(512, 64)

    feat = pl.pallas_call(
        _conv_stack_kernel,
        out_shape=jax.ShapeDtypeStruct((N, 49, 64), jnp.float32),
        grid=(N // _B_BLK,),
        in_specs=[
            pl.BlockSpec((_B_BLK, 21, 21, 64), lambda i: (i, 0, 0, 0)),
            pl.BlockSpec((256, 32), lambda i: (0, 0)),
            pl.BlockSpec((1, 32), lambda i: (0, 0)),
            pl.BlockSpec((512, 64), lambda i: (0, 0)),
            pl.BlockSpec((1, 64), lambda i: (0, 0)),
            pl.BlockSpec((576, 64), lambda i: (0, 0)),
            pl.BlockSpec((1, 64), lambda i: (0, 0)),
        ],
        out_specs=pl.BlockSpec((_B_BLK, 49, 64), lambda i: (i, 0, 0)),
        compiler_params=pltpu.CompilerParams(
            dimension_semantics=("parallel",)),
    )(xs, w1p, c1_b, w2p, c2_b, c3_w, c3_b)

    feat = feat.reshape(N, 49 * 64)

    tm = 128
    return pl.pallas_call(
        _fc_head_kernel,
        out_shape=jax.ShapeDtypeStruct((N, 18), jnp.float32),
        grid=(N // tm,),
        in_specs=[
            pl.BlockSpec((tm, 3136), lambda i: (i, 0)),
            pl.BlockSpec((3136, 512), lambda i: (0, 0)),
            pl.BlockSpec((1, 512), lambda i: (0, 0)),
            pl.BlockSpec((512, 18), lambda i: (0, 0)),
            pl.BlockSpec((1, 18), lambda i: (0, 0)),
        ],
        out_specs=pl.BlockSpec((tm, 18), lambda i: (i, 0)),
        compiler_params=pltpu.CompilerParams(
            dimension_semantics=("parallel",)),
    )(feat, fc1_w, fc1_b, fc2_w, fc2_b)
